# baseline (speedup 1.0000x reference)
import jax
import jax.numpy as jnp
from jax.experimental import pallas as pl
from jax.experimental.pallas import tpu as pltpu


def _cls_body(p_ref, w_ref, b_ref, o_ref):
    o_ref[...] = jnp.dot(p_ref[...], w_ref[...],
                         preferred_element_type=jnp.float32) + b_ref[...]


def _cgconv(x, src, dst, edge_attr, Wf, bf, Ws, bs):
    x_i = jnp.take(x, dst, axis=0)
    x_j = jnp.take(x, src, axis=0)
    z = jnp.concatenate([x_i, x_j, edge_attr], axis=1)
    msg = jax.nn.sigmoid(z @ Wf + bf) * jax.nn.softplus(z @ Ws + bs)
    agg = jax.ops.segment_sum(msg, dst, num_segments=x.shape[0])
    return x + agg


def kernel(x, edge_index, edge_attr, batch,
           Wf1, bf1, Ws1, bs1, Wf2, bf2, Ws2, bs2, Wf3, bf3, Ws3, bs3,
           lin_W, lin_b, cls_W, cls_b):
    G = 128
    src = edge_index[0]
    dst = edge_index[1]
    h = _cgconv(x, src, dst, edge_attr, Wf1, bf1, Ws1, bs1)
    x1 = h @ lin_W + lin_b
    h = _cgconv(h, src, dst, edge_attr, Wf2, bf2, Ws2, bs2)
    x2 = h @ lin_W + lin_b
    h = _cgconv(h, src, dst, edge_attr, Wf3, bf3, Ws3, bs3)
    x3 = h @ lin_W + lin_b
    p1 = jax.ops.segment_max(x1, batch, num_segments=G)
    p2 = jax.ops.segment_max(x2, batch, num_segments=G)
    p3 = jax.ops.segment_max(x3, batch, num_segments=G)
    pooled = p1 + p2 + p3

    ncls = cls_W.shape[1]
    wpad = jnp.pad(cls_W, ((0, 0), (0, 128 - ncls)))
    bpad = jnp.pad(cls_b, (0, 128 - ncls)).reshape(1, 128)
    out = pl.pallas_call(
        _cls_body,
        out_shape=jax.ShapeDtypeStruct((G, 128), jnp.float32),
    )(pooled, wpad, bpad)
    return out[:, :ncls]


# speedup vs baseline: 21.0319x; 21.0319x over previous
import jax
import jax.numpy as jnp
from jax import lax
from jax.experimental import pallas as pl
from jax.experimental.pallas import tpu as pltpu
from jax.experimental.pallas import tpu_sc as plsc

N = 100000
NP = 100352
E = 6400000
G = 128
HID = 128
NB = E // 128
NC = 2
NS = 16
NW = NC * NS
RPT = NP // NS
RB = 2048
NBK = NP // RB


def _conv_body(h0_hbm, h1_hbm, h2_hbm, ei, ea, wpk, zro,
               o0, o1, o2, o3, o4, o5,
               hs0, hs1, hs2, ag0, ag1, ag2, wv,
               idxb, eab, xd0, xd1, xd2, xs0, xs1, xs2, m0, m1, m2,
               s0, s1, s2, s3, s4, s5):
    cid = lax.axis_index("c")
    sid = lax.axis_index("s")
    wid = sid * NC + cid
    r0 = sid * RPT
    for hbm, sh in ((h0_hbm, hs0), (h1_hbm, hs1), (h2_hbm, hs2),
                    (zro, ag0), (zro, ag1), (zro, ag2)):
        pltpu.sync_copy(hbm.at[pl.ds(r0, RPT)], sh.at[pl.ds(r0, RPT)])
    pltpu.sync_copy(wpk, wv)
    plsc.subcore_barrier()

    wf = [[wv[j * 3 + k] for k in range(3)] for j in range(10)]
    bf = [wv[30 + k] for k in range(3)]
    ws = [[wv[33 + j * 3 + k] for k in range(3)] for j in range(10)]
    bs = [wv[63 + k] for k in range(3)]

    b0 = (NB * wid) // NW
    b1 = (NB * (wid + 1)) // NW

    def block_body(b, carry):
        pltpu.sync_copy(ei.at[b], idxb)
        pltpu.sync_copy(ea.at[b], eab)
        cps = [pltpu.async_copy(hs0.at[idxb.at[1]], xd0, s0),
               pltpu.async_copy(hs1.at[idxb.at[1]], xd1, s1),
               pltpu.async_copy(hs2.at[idxb.at[1]], xd2, s2),
               pltpu.async_copy(hs0.at[idxb.at[0]], xs0, s3),
               pltpu.async_copy(hs1.at[idxb.at[0]], xs1, s4),
               pltpu.async_copy(hs2.at[idxb.at[0]], xs2, s5)]
        for c in cps:
            c.wait()
        for g in range(8):
            sl = pl.ds(g * 16, 16)
            ins = [xd0[sl], xd1[sl], xd2[sl], xs0[sl], xs1[sl], xs2[sl],
                   eab[0, sl], eab[1, sl], eab[2, sl], eab[3, sl]]
            for k, mref in enumerate((m0, m1, m2)):
                gate = bf[k]
                core = bs[k]
                for j in range(10):
                    gate = gate + ins[j] * wf[j][k]
                    core = core + ins[j] * ws[j][k]
                sig = 1.0 / (1.0 + jnp.exp(-gate))
                a = jnp.abs(core)
                e = jnp.exp(-a)
                t = e / (e + 2.0)
                t2 = t * t
                p = t2 * (1 / 3 + t2 * (1 / 5 + t2 * (1 / 7 + t2 * (1 / 9))))
                sp = jnp.maximum(core, 0.0) + 2.0 * t * (1.0 + p)
                mref[sl] = sig * sp
        pltpu.sync_copy(m0, ag0.at[idxb.at[1]], add=True)
        pltpu.sync_copy(m1, ag1.at[idxb.at[1]], add=True)
        pltpu.sync_copy(m2, ag2.at[idxb.at[1]], add=True)
        return carry

    lax.fori_loop(b0, b1, block_body, 0)
    plsc.subcore_barrier()

    @pl.when(cid == 0)
    def _():
        for ag, o in ((ag0, o0), (ag1, o1), (ag2, o2)):
            pltpu.sync_copy(ag.at[pl.ds(r0, RPT)], o.at[pl.ds(r0, RPT)])

    @pl.when(cid == 1)
    def _():
        for ag, o in ((ag0, o3), (ag1, o4), (ag2, o5)):
            pltpu.sync_copy(ag.at[pl.ds(r0, RPT)], o.at[pl.ds(r0, RPT)])


_conv_sc = pl.kernel(
    _conv_body,
    out_type=[jax.ShapeDtypeStruct((NP,), jnp.float32)] * 6,
    mesh=plsc.VectorSubcoreMesh(core_axis_name="c", subcore_axis_name="s"),
    scratch_types=[
        pltpu.VMEM_SHARED((NP,), jnp.float32),
        pltpu.VMEM_SHARED((NP,), jnp.float32),
        pltpu.VMEM_SHARED((NP,), jnp.float32),
        pltpu.VMEM_SHARED((NP,), jnp.float32),
        pltpu.VMEM_SHARED((NP,), jnp.float32),
        pltpu.VMEM_SHARED((NP,), jnp.float32),
        pltpu.VMEM((66, 16), jnp.float32),
        pltpu.VMEM((2, 128), jnp.int32),
        pltpu.VMEM((4, 128), jnp.float32),
        pltpu.VMEM((128,), jnp.float32),
        pltpu.VMEM((128,), jnp.float32),
        pltpu.VMEM((128,), jnp.float32),
        pltpu.VMEM((128,), jnp.float32),
        pltpu.VMEM((128,), jnp.float32),
        pltpu.VMEM((128,), jnp.float32),
        pltpu.VMEM((128,), jnp.float32),
        pltpu.VMEM((128,), jnp.float32),
        pltpu.VMEM((128,), jnp.float32),
        pltpu.SemaphoreType.DMA,
        pltpu.SemaphoreType.DMA,
        pltpu.SemaphoreType.DMA,
        pltpu.SemaphoreType.DMA,
        pltpu.SemaphoreType.DMA,
        pltpu.SemaphoreType.DMA,
    ],
)


def _pool_body(h0, h1, h2, a00, a01, a02, a10, a11, a12, wT, bc, bt,
               hn0, hn1, hn2, p_ref, acc_ref):
    i = pl.program_id(0)

    @pl.when(i == 0)
    def _():
        acc_ref[...] = jnp.full((HID, G), -jnp.inf, jnp.float32)

    hn = []
    for hr, a0r, a1r, hnr in ((h0, a00, a10, hn0), (h1, a01, a11, hn1),
                              (h2, a02, a12, hn2)):
        v = hr[...] + a0r[...] + a1r[...]
        hnr[...] = v
        hn.append(v)
    y = bc[...] + (wT[:, 0:1] * hn[0] + wT[:, 1:2] * hn[1]
                   + wT[:, 2:3] * hn[2])
    btv = bt[...]
    g0 = jnp.min(btv)
    g1 = jnp.max(btv)
    lanes = lax.broadcasted_iota(jnp.int32, (HID, G), 1)

    def upd(g, carry):
        m = jnp.max(jnp.where(btv == g, y, -jnp.inf), axis=1)
        acc_ref[...] = jnp.maximum(
            acc_ref[...], jnp.where(lanes == g, m[:, None], -jnp.inf))
        return carry

    lax.fori_loop(g0, g1 + 1, upd, 0)

    @pl.when(i == NBK - 1)
    def _():
        p_ref[...] = acc_ref[...]


_pool_tc = pl.pallas_call(
    _pool_body,
    grid=(NBK,),
    in_specs=[pl.BlockSpec((1, RB), lambda i: (0, i)) for _ in range(9)] + [
        pl.BlockSpec((HID, 3), lambda i: (0, 0)),
        pl.BlockSpec((HID, 1), lambda i: (0, 0)),
        pl.BlockSpec((1, RB), lambda i: (0, i)),
    ],
    out_specs=[pl.BlockSpec((1, RB), lambda i: (0, i)) for _ in range(3)] + [
        pl.BlockSpec((HID, G), lambda i: (0, 0)),
    ],
    out_shape=[jax.ShapeDtypeStruct((1, NP), jnp.float32)] * 3 + [
        jax.ShapeDtypeStruct((HID, G), jnp.float32),
    ],
    scratch_shapes=[pltpu.VMEM((HID, G), jnp.float32)],
)


def _cls_body(p_ref, w_ref, b_ref, o_ref):
    o_ref[...] = jnp.dot(p_ref[...], w_ref[...],
                         preferred_element_type=jnp.float32) + b_ref[...]


def kernel(x, edge_index, edge_attr, batch,
           Wf1, bf1, Ws1, bs1, Wf2, bf2, Ws2, bs2, Wf3, bf3, Ws3, bs3,
           lin_W, lin_b, cls_W, cls_b):
    hp = [jnp.pad(x[:, j], (0, NP - N)) for j in range(3)]
    ei = edge_index.reshape(2, NB, 128).transpose(1, 0, 2)
    ea = edge_attr.reshape(NB, 128, 4).transpose(0, 2, 1)
    zro = jnp.zeros((NP,), jnp.float32)
    bt = jnp.pad(batch, (0, NP - N), constant_values=G).reshape(1, NP)
    wT = lin_W.T
    bc = lin_b.reshape(HID, 1)

    pools = []
    for Wf, bfv, Ws, bsv in ((Wf1, bf1, Ws1, bs1), (Wf2, bf2, Ws2, bs2),
                             (Wf3, bf3, Ws3, bs3)):
        wpk = jnp.concatenate([Wf.ravel(), bfv, Ws.ravel(), bsv])
        wpk = jnp.tile(wpk[:, None], (1, 16))
        o = _conv_sc(hp[0], hp[1], hp[2], ei, ea, wpk, zro)
        args = [h.reshape(1, NP) for h in hp]
        args += [o[j].reshape(1, NP) for j in range(6)]
        hn0, hn1, hn2, p = _pool_tc(*args, wT, bc, bt)
        hp = [h.reshape(NP) for h in (hn0, hn1, hn2)]
        pools.append(p)
    pooled = (pools[0] + pools[1] + pools[2]).T

    ncls = cls_W.shape[1]
    wpad = jnp.pad(cls_W, ((0, 0), (0, 128 - ncls)))
    bpad = jnp.pad(cls_b, (0, 128 - ncls)).reshape(1, 128)
    out = pl.pallas_call(
        _cls_body,
        out_shape=jax.ShapeDtypeStruct((G, 128), jnp.float32),
    )(pooled, wpad, bpad)
    return out[:, :ncls]


# speedup vs baseline: 21.5680x; 1.0255x over previous
import jax
import jax.numpy as jnp
from jax import lax
from jax.experimental import pallas as pl
from jax.experimental.pallas import tpu as pltpu
from jax.experimental.pallas import tpu_sc as plsc

N = 100000
NP = 100352
E = 6400000
G = 128
HID = 128
NB = E // 128
K = 4
NQ = NB // K
NC = 2
NS = 16
NW = NC * NS
RPT = NP // NS
RB = 2048
NBK = NP // RB


def _conv_body(h0_hbm, h1_hbm, h2_hbm, pks, pkd, pka, wpk, zro,
               o0, o1, o2, o3, o4, o5,
               hs0, hs1, hs2, ag0, ag1, ag2, wv,
               stbs, stbd, eab, sidx, xd0, xd1, xd2, xs0, xs1, xs2,
               m0, m1, m2, sg, ss):
    cid = lax.axis_index("c")
    sid = lax.axis_index("s")
    wid = sid * NC + cid
    r0 = sid * RPT
    for hbm, sh in ((h0_hbm, hs0), (h1_hbm, hs1), (h2_hbm, hs2),
                    (zro, ag0), (zro, ag1), (zro, ag2)):
        pltpu.sync_copy(hbm.at[pl.ds(r0, RPT)], sh.at[pl.ds(r0, RPT)])
    pltpu.sync_copy(wpk, wv)
    plsc.subcore_barrier()

    wf = [[wv[j * 3 + k] for k in range(3)] for j in range(10)]
    bf = [wv[30 + k] for k in range(3)]
    ws = [[wv[33 + j * 3 + k] for k in range(3)] for j in range(10)]
    bs = [wv[63 + k] for k in range(3)]

    q0 = (NQ * wid) // NW
    q1 = (NQ * (wid + 1)) // NW

    def gathers(kk):
        idxd = stbd.at[kk]
        idxs = stbs.at[kk]
        dsl = pl.ds(128 * kk, 128)
        return [pltpu.async_copy(hs0.at[idxd], xd0.at[dsl], sg),
                pltpu.async_copy(hs1.at[idxd], xd1.at[dsl], sg),
                pltpu.async_copy(hs2.at[idxd], xd2.at[dsl], sg),
                pltpu.async_copy(hs0.at[idxs], xs0.at[dsl], sg),
                pltpu.async_copy(hs1.at[idxs], xs1.at[dsl], sg),
                pltpu.async_copy(hs2.at[idxs], xs2.at[dsl], sg)]

    def scatters(kk, start):
        idx = sidx.at[kk]
        dsl = pl.ds(128 * kk, 128)
        if start:
            return [pltpu.async_copy(m0.at[dsl], ag0.at[idx], ss, add=True),
                    pltpu.async_copy(m1.at[dsl], ag1.at[idx], ss, add=True),
                    pltpu.async_copy(m2.at[dsl], ag2.at[idx], ss, add=True)]
        return [pltpu.make_async_copy(m0.at[dsl], ag0.at[idx], ss),
                pltpu.make_async_copy(m1.at[dsl], ag1.at[idx], ss),
                pltpu.make_async_copy(m2.at[dsl], ag2.at[idx], ss)]

    def quad_body(q, carry):
        pltpu.sync_copy(pks.at[q], stbs)
        pltpu.sync_copy(pkd.at[q], stbd)
        pltpu.sync_copy(pka.at[q], eab)
        gcps = [c for kk in range(K) for c in gathers(kk)]
        @pl.when(q > q0)
        def _():
            for kk in range(K):
                for c in scatters(kk, start=False):
                    c.wait()
        pltpu.sync_copy(pkd.at[q], sidx)
        for c in gcps:
            c.wait()
        for kk in range(K):
            for g in range(8):
                sl = pl.ds(kk * 128 + g * 16, 16)
                st = pl.ds(g * 16, 16)
                ins = [xd0[sl], xd1[sl], xd2[sl], xs0[sl], xs1[sl], xs2[sl]]
                ins += [eab[kk, j, st] for j in range(4)]
                for k, mref in enumerate((m0, m1, m2)):
                    gate = bf[k]
                    core = bs[k]
                    for j in range(10):
                        gate = gate + ins[j] * wf[j][k]
                        core = core + ins[j] * ws[j][k]
                    sig = 1.0 / (1.0 + jnp.exp(-gate))
                    a = jnp.abs(core)
                    e = jnp.exp(-a)
                    t = e / (e + 2.0)
                    t2 = t * t
                    p = t2 * (1 / 3 + t2 * (1 / 5
                              + t2 * (1 / 7 + t2 * (1 / 9))))
                    sp = jnp.maximum(core, 0.0) + 2.0 * t * (1.0 + p)
                    mref[sl] = sig * sp
        for kk in range(K):
            scatters(kk, start=True)
        return carry

    lax.fori_loop(q0, q1, quad_body, 0)

    @pl.when(q1 > q0)
    def _():
        for kk in range(K):
            for c in scatters(kk, start=False):
                c.wait()

    plsc.subcore_barrier()

    @pl.when(cid == 0)
    def _():
        for ag, o in ((ag0, o0), (ag1, o1), (ag2, o2)):
            pltpu.sync_copy(ag.at[pl.ds(r0, RPT)], o.at[pl.ds(r0, RPT)])

    @pl.when(cid == 1)
    def _():
        for ag, o in ((ag0, o3), (ag1, o4), (ag2, o5)):
            pltpu.sync_copy(ag.at[pl.ds(r0, RPT)], o.at[pl.ds(r0, RPT)])


_conv_sc = pl.kernel(
    _conv_body,
    out_type=[jax.ShapeDtypeStruct((NP,), jnp.float32)] * 6,
    mesh=plsc.VectorSubcoreMesh(core_axis_name="c", subcore_axis_name="s"),
    scratch_types=[
        pltpu.VMEM_SHARED((NP,), jnp.float32),
        pltpu.VMEM_SHARED((NP,), jnp.float32),
        pltpu.VMEM_SHARED((NP,), jnp.float32),
        pltpu.VMEM_SHARED((NP,), jnp.float32),
        pltpu.VMEM_SHARED((NP,), jnp.float32),
        pltpu.VMEM_SHARED((NP,), jnp.float32),
        pltpu.VMEM((66, 16), jnp.float32),
        pltpu.VMEM((K, 128), jnp.int32),
        pltpu.VMEM((K, 128), jnp.int32),
        pltpu.VMEM((K, 4, 128), jnp.float32),
        pltpu.VMEM((K, 128), jnp.int32),
        pltpu.VMEM((K * 128,), jnp.float32),
        pltpu.VMEM((K * 128,), jnp.float32),
        pltpu.VMEM((K * 128,), jnp.float32),
        pltpu.VMEM((K * 128,), jnp.float32),
        pltpu.VMEM((K * 128,), jnp.float32),
        pltpu.VMEM((K * 128,), jnp.float32),
        pltpu.VMEM((K * 128,), jnp.float32),
        pltpu.VMEM((K * 128,), jnp.float32),
        pltpu.VMEM((K * 128,), jnp.float32),
        pltpu.SemaphoreType.DMA,
        pltpu.SemaphoreType.DMA,
    ],
)


def _pool_body(h0, h1, h2, a00, a01, a02, a10, a11, a12, wT, bc, bt,
               hn0, hn1, hn2, p_ref, acc_ref):
    i = pl.program_id(0)

    @pl.when(i == 0)
    def _():
        acc_ref[...] = jnp.full((HID, G), -jnp.inf, jnp.float32)

    hn = []
    for hr, a0r, a1r, hnr in ((h0, a00, a10, hn0), (h1, a01, a11, hn1),
                              (h2, a02, a12, hn2)):
        v = hr[...] + a0r[...] + a1r[...]
        hnr[...] = v
        hn.append(v)
    y = bc[...] + (wT[:, 0:1] * hn[0] + wT[:, 1:2] * hn[1]
                   + wT[:, 2:3] * hn[2])
    btv = bt[...]
    g0 = jnp.min(btv)
    g1 = jnp.max(btv)
    lanes = lax.broadcasted_iota(jnp.int32, (HID, G), 1)

    def upd(g, carry):
        m = jnp.max(jnp.where(btv == g, y, -jnp.inf), axis=1)
        acc_ref[...] = jnp.maximum(
            acc_ref[...], jnp.where(lanes == g, m[:, None], -jnp.inf))
        return carry

    lax.fori_loop(g0, g1 + 1, upd, 0)

    @pl.when(i == NBK - 1)
    def _():
        p_ref[...] = acc_ref[...]


_pool_tc = pl.pallas_call(
    _pool_body,
    grid=(NBK,),
    in_specs=[pl.BlockSpec((1, RB), lambda i: (0, i)) for _ in range(9)] + [
        pl.BlockSpec((HID, 3), lambda i: (0, 0)),
        pl.BlockSpec((HID, 1), lambda i: (0, 0)),
        pl.BlockSpec((1, RB), lambda i: (0, i)),
    ],
    out_specs=[pl.BlockSpec((1, RB), lambda i: (0, i)) for _ in range(3)] + [
        pl.BlockSpec((HID, G), lambda i: (0, 0)),
    ],
    out_shape=[jax.ShapeDtypeStruct((1, NP), jnp.float32)] * 3 + [
        jax.ShapeDtypeStruct((HID, G), jnp.float32),
    ],
    scratch_shapes=[pltpu.VMEM((HID, G), jnp.float32)],
)


def _cls_body(p_ref, w_ref, b_ref, o_ref):
    o_ref[...] = jnp.dot(p_ref[...], w_ref[...],
                         preferred_element_type=jnp.float32) + b_ref[...]


def kernel(x, edge_index, edge_attr, batch,
           Wf1, bf1, Ws1, bs1, Wf2, bf2, Ws2, bs2, Wf3, bf3, Ws3, bs3,
           lin_W, lin_b, cls_W, cls_b):
    hp = [jnp.pad(x[:, j], (0, NP - N)) for j in range(3)]
    pks = edge_index[0].reshape(NQ, K, 128)
    pkd = edge_index[1].reshape(NQ, K, 128)
    pka = edge_attr.reshape(NB, 128, 4).transpose(0, 2, 1)
    pka = pka.reshape(NQ, K, 4, 128)
    zro = jnp.zeros((NP,), jnp.float32)
    bt = jnp.pad(batch, (0, NP - N), constant_values=G).reshape(1, NP)
    wT = lin_W.T
    bc = lin_b.reshape(HID, 1)

    pools = []
    for Wf, bfv, Ws, bsv in ((Wf1, bf1, Ws1, bs1), (Wf2, bf2, Ws2, bs2),
                             (Wf3, bf3, Ws3, bs3)):
        wpk = jnp.concatenate([Wf.ravel(), bfv, Ws.ravel(), bsv])
        wpk = jnp.tile(wpk[:, None], (1, 16))
        o = _conv_sc(hp[0], hp[1], hp[2], pks, pkd, pka, wpk, zro)
        args = [h.reshape(1, NP) for h in hp]
        args += [o[j].reshape(1, NP) for j in range(6)]
        hn0, hn1, hn2, p = _pool_tc(*args, wT, bc, bt)
        hp = [h.reshape(NP) for h in (hn0, hn1, hn2)]
        pools.append(p)
    pooled = (pools[0] + pools[1] + pools[2]).T

    ncls = cls_W.shape[1]
    wpad = jnp.pad(cls_W, ((0, 0), (0, 128 - ncls)))
    bpad = jnp.pad(cls_b, (0, 128 - ncls)).reshape(1, 128)
    out = pl.pallas_call(
        _cls_body,
        out_shape=jax.ShapeDtypeStruct((G, 128), jnp.float32),
    )(pooled, wpad, bpad)
    return out[:, :ncls]


# speedup vs baseline: 21.8021x; 1.0109x over previous
import jax
import jax.numpy as jnp
from jax import lax
from jax.experimental import pallas as pl
from jax.experimental.pallas import tpu as pltpu
from jax.experimental.pallas import tpu_sc as plsc

N = 100000
NP = 100352
E = 6400000
G = 128
HID = 128
NB = E // 128
K = 4
NQ = NB // K
NC = 2
NS = 16
NW = NC * NS
RPT = NP // NS
RB = 2048
NBK = NP // RB


def _conv_body(h0_hbm, h1_hbm, h2_hbm, pks, pkd, pka, wpk, zro,
               o0, o1, o2, o3, o4, o5,
               hs0, hs1, hs2, ag0, ag1, ag2, wv,
               stbs, stbd, eab, sidx, xd0, xd1, xd2, xs0, xs1, xs2,
               m0, m1, m2, sg0, sg1, sg2, sg3, ss):
    cid = lax.axis_index("c")
    sid = lax.axis_index("s")
    wid = sid * NC + cid
    r0 = sid * RPT
    for hbm, sh in ((h0_hbm, hs0), (h1_hbm, hs1), (h2_hbm, hs2),
                    (zro, ag0), (zro, ag1), (zro, ag2)):
        pltpu.sync_copy(hbm.at[pl.ds(r0, RPT)], sh.at[pl.ds(r0, RPT)])
    pltpu.sync_copy(wpk, wv)
    plsc.subcore_barrier()

    wf = [[wv[j * 3 + k] for k in range(3)] for j in range(10)]
    bf = [wv[30 + k] for k in range(3)]
    ws = [[wv[33 + j * 3 + k] for k in range(3)] for j in range(10)]
    bs = [wv[63 + k] for k in range(3)]

    q0 = (NQ * wid) // NW
    q1 = (NQ * (wid + 1)) // NW

    sgs = (sg0, sg1, sg2, sg3)

    def gathers(kk):
        idxd = stbd.at[kk]
        idxs = stbs.at[kk]
        dsl = pl.ds(128 * kk, 128)
        sg = sgs[kk]
        return [pltpu.async_copy(hs0.at[idxd], xd0.at[dsl], sg),
                pltpu.async_copy(hs1.at[idxd], xd1.at[dsl], sg),
                pltpu.async_copy(hs2.at[idxd], xd2.at[dsl], sg),
                pltpu.async_copy(hs0.at[idxs], xs0.at[dsl], sg),
                pltpu.async_copy(hs1.at[idxs], xs1.at[dsl], sg),
                pltpu.async_copy(hs2.at[idxs], xs2.at[dsl], sg)]

    def scatters(kk, start):
        idx = sidx.at[kk]
        dsl = pl.ds(128 * kk, 128)
        if start:
            return [pltpu.async_copy(m0.at[dsl], ag0.at[idx], ss, add=True),
                    pltpu.async_copy(m1.at[dsl], ag1.at[idx], ss, add=True),
                    pltpu.async_copy(m2.at[dsl], ag2.at[idx], ss, add=True)]
        return [pltpu.make_async_copy(m0.at[dsl], ag0.at[idx], ss),
                pltpu.make_async_copy(m1.at[dsl], ag1.at[idx], ss),
                pltpu.make_async_copy(m2.at[dsl], ag2.at[idx], ss)]

    def quad_body(q, carry):
        pltpu.sync_copy(pks.at[q], stbs)
        pltpu.sync_copy(pkd.at[q], stbd)
        pltpu.sync_copy(pka.at[q], eab)
        gcls = [gathers(kk) for kk in range(K)]
        @pl.when(q > q0)
        def _():
            for kk in range(K):
                for c in scatters(kk, start=False):
                    c.wait()
        pltpu.sync_copy(pkd.at[q], sidx)
        for kk in range(K):
            for c in gcls[kk]:
                c.wait()
            for g in range(8):
                sl = pl.ds(kk * 128 + g * 16, 16)
                st = pl.ds(g * 16, 16)
                ins = [xd0[sl], xd1[sl], xd2[sl], xs0[sl], xs1[sl], xs2[sl]]
                ins += [eab[kk, j, st] for j in range(4)]
                for k, mref in enumerate((m0, m1, m2)):
                    gate = bf[k]
                    core = bs[k]
                    for j in range(10):
                        gate = gate + ins[j] * wf[j][k]
                        core = core + ins[j] * ws[j][k]
                    sig = 1.0 / (1.0 + jnp.exp(-gate))
                    a = jnp.abs(core)
                    e = jnp.exp(-a)
                    t = e / (e + 2.0)
                    t2 = t * t
                    p = t2 * (1 / 3 + t2 * (1 / 5
                              + t2 * (1 / 7 + t2 * (1 / 9))))
                    sp = jnp.maximum(core, 0.0) + 2.0 * t * (1.0 + p)
                    mref[sl] = sig * sp
            scatters(kk, start=True)
        return carry

    lax.fori_loop(q0, q1, quad_body, 0)

    @pl.when(q1 > q0)
    def _():
        for kk in range(K):
            for c in scatters(kk, start=False):
                c.wait()

    plsc.subcore_barrier()

    @pl.when(cid == 0)
    def _():
        for ag, o in ((ag0, o0), (ag1, o1), (ag2, o2)):
            pltpu.sync_copy(ag.at[pl.ds(r0, RPT)], o.at[pl.ds(r0, RPT)])

    @pl.when(cid == 1)
    def _():
        for ag, o in ((ag0, o3), (ag1, o4), (ag2, o5)):
            pltpu.sync_copy(ag.at[pl.ds(r0, RPT)], o.at[pl.ds(r0, RPT)])


_conv_sc = pl.kernel(
    _conv_body,
    out_type=[jax.ShapeDtypeStruct((NP,), jnp.float32)] * 6,
    mesh=plsc.VectorSubcoreMesh(core_axis_name="c", subcore_axis_name="s"),
    scratch_types=[
        pltpu.VMEM_SHARED((NP,), jnp.float32),
        pltpu.VMEM_SHARED((NP,), jnp.float32),
        pltpu.VMEM_SHARED((NP,), jnp.float32),
        pltpu.VMEM_SHARED((NP,), jnp.float32),
        pltpu.VMEM_SHARED((NP,), jnp.float32),
        pltpu.VMEM_SHARED((NP,), jnp.float32),
        pltpu.VMEM((66, 16), jnp.float32),
        pltpu.VMEM((K, 128), jnp.int32),
        pltpu.VMEM((K, 128), jnp.int32),
        pltpu.VMEM((K, 4, 128), jnp.float32),
        pltpu.VMEM((K, 128), jnp.int32),
        pltpu.VMEM((K * 128,), jnp.float32),
        pltpu.VMEM((K * 128,), jnp.float32),
        pltpu.VMEM((K * 128,), jnp.float32),
        pltpu.VMEM((K * 128,), jnp.float32),
        pltpu.VMEM((K * 128,), jnp.float32),
        pltpu.VMEM((K * 128,), jnp.float32),
        pltpu.VMEM((K * 128,), jnp.float32),
        pltpu.VMEM((K * 128,), jnp.float32),
        pltpu.VMEM((K * 128,), jnp.float32),
        pltpu.SemaphoreType.DMA,
        pltpu.SemaphoreType.DMA,
        pltpu.SemaphoreType.DMA,
        pltpu.SemaphoreType.DMA,
        pltpu.SemaphoreType.DMA,
    ],
)


def _pool_body(h0, h1, h2, a00, a01, a02, a10, a11, a12, wT, bc, bt,
               hn0, hn1, hn2, p_ref, acc_ref):
    i = pl.program_id(0)

    @pl.when(i == 0)
    def _():
        acc_ref[...] = jnp.full((HID, G), -jnp.inf, jnp.float32)

    hn = []
    for hr, a0r, a1r, hnr in ((h0, a00, a10, hn0), (h1, a01, a11, hn1),
                              (h2, a02, a12, hn2)):
        v = hr[...] + a0r[...] + a1r[...]
        hnr[...] = v
        hn.append(v)
    y = bc[...] + (wT[:, 0:1] * hn[0] + wT[:, 1:2] * hn[1]
                   + wT[:, 2:3] * hn[2])
    btv = bt[...]
    g0 = jnp.min(btv)
    g1 = jnp.max(btv)
    lanes = lax.broadcasted_iota(jnp.int32, (HID, G), 1)

    def upd(g, carry):
        m = jnp.max(jnp.where(btv == g, y, -jnp.inf), axis=1)
        acc_ref[...] = jnp.maximum(
            acc_ref[...], jnp.where(lanes == g, m[:, None], -jnp.inf))
        return carry

    lax.fori_loop(g0, g1 + 1, upd, 0)

    @pl.when(i == NBK - 1)
    def _():
        p_ref[...] = acc_ref[...]


_pool_tc = pl.pallas_call(
    _pool_body,
    grid=(NBK,),
    in_specs=[pl.BlockSpec((1, RB), lambda i: (0, i)) for _ in range(9)] + [
        pl.BlockSpec((HID, 3), lambda i: (0, 0)),
        pl.BlockSpec((HID, 1), lambda i: (0, 0)),
        pl.BlockSpec((1, RB), lambda i: (0, i)),
    ],
    out_specs=[pl.BlockSpec((1, RB), lambda i: (0, i)) for _ in range(3)] + [
        pl.BlockSpec((HID, G), lambda i: (0, 0)),
    ],
    out_shape=[jax.ShapeDtypeStruct((1, NP), jnp.float32)] * 3 + [
        jax.ShapeDtypeStruct((HID, G), jnp.float32),
    ],
    scratch_shapes=[pltpu.VMEM((HID, G), jnp.float32)],
)


def _cls_body(p_ref, w_ref, b_ref, o_ref):
    o_ref[...] = jnp.dot(p_ref[...], w_ref[...],
                         preferred_element_type=jnp.float32) + b_ref[...]


def kernel(x, edge_index, edge_attr, batch,
           Wf1, bf1, Ws1, bs1, Wf2, bf2, Ws2, bs2, Wf3, bf3, Ws3, bs3,
           lin_W, lin_b, cls_W, cls_b):
    hp = [jnp.pad(x[:, j], (0, NP - N)) for j in range(3)]
    pks = edge_index[0].reshape(NQ, K, 128)
    pkd = edge_index[1].reshape(NQ, K, 128)
    pka = edge_attr.reshape(NB, 128, 4).transpose(0, 2, 1)
    pka = pka.reshape(NQ, K, 4, 128)
    zro = jnp.zeros((NP,), jnp.float32)
    bt = jnp.pad(batch, (0, NP - N), constant_values=G).reshape(1, NP)
    wT = lin_W.T
    bc = lin_b.reshape(HID, 1)

    pools = []
    for Wf, bfv, Ws, bsv in ((Wf1, bf1, Ws1, bs1), (Wf2, bf2, Ws2, bs2),
                             (Wf3, bf3, Ws3, bs3)):
        wpk = jnp.concatenate([Wf.ravel(), bfv, Ws.ravel(), bsv])
        wpk = jnp.tile(wpk[:, None], (1, 16))
        o = _conv_sc(hp[0], hp[1], hp[2], pks, pkd, pka, wpk, zro)
        args = [h.reshape(1, NP) for h in hp]
        args += [o[j].reshape(1, NP) for j in range(6)]
        hn0, hn1, hn2, p = _pool_tc(*args, wT, bc, bt)
        hp = [h.reshape(NP) for h in (hn0, hn1, hn2)]
        pools.append(p)
    pooled = (pools[0] + pools[1] + pools[2]).T

    ncls = cls_W.shape[1]
    wpad = jnp.pad(cls_W, ((0, 0), (0, 128 - ncls)))
    bpad = jnp.pad(cls_b, (0, 128 - ncls)).reshape(1, 128)
    out = pl.pallas_call(
        _cls_body,
        out_shape=jax.ShapeDtypeStruct((G, 128), jnp.float32),
    )(pooled, wpad, bpad)
    return out[:, :ncls]
